# baseline (speedup 1.0000x reference)
import functools

import jax
import jax.numpy as jnp
from jax import lax
from jax.experimental import pallas as pl
from jax.experimental.pallas import tpu as pltpu
from jax.experimental.pallas import tpu_sc as plsc

N = 10000
E = 320000
D = 128

NC = 2
NS = 16
NW = NC * NS
C = 128
CPW = 80
CPW0 = 96
CPW1 = 64
E_PAD = NW * CPW * C
N_PAD = 10112
RPT = N_PAD // NS

_MESH = plsc.VectorSubcoreMesh(core_axis_name="c", subcore_axis_name="s")


def _zero_fill(ref, val):
  rows, cols = ref.shape
  def body(i, carry):
    ref[i // (cols // 16), pl.ds((i % (cols // 16)) * 16, 16)] = (
        jnp.full((16,), val, jnp.float32))
    return carry
  lax.fori_loop(0, rows * (cols // 16), body, 0)


def _zero_spmem_slice(zsrc_v, sp, base):
  for k in range(RPT // C):
    pltpu.sync_copy(zsrc_v, sp.at[pl.ds(base + k * C, C)])
  rem = RPT % C
  if rem:
    pltpu.sync_copy(zsrc_v.at[pl.ds(0, rem)],
                    sp.at[pl.ds(base + (RPT // C) * C, rem)])


def _sc_segsum_body(table_hbm, iv_hbm, out_sum,
                    ib0, ib1, rows_a, rows_b,
                    psum_sp, gs_a, gs_b, ss_a, ss_b, is0, is1):
  cid = lax.axis_index("c")
  sid = lax.axis_index("s")
  base = sid * RPT

  _zero_fill(rows_a, 0.0)
  _zero_spmem_slice(rows_a, psum_sp, base)

  plsc.subcore_barrier()

  nq = jnp.where(cid == 0, CPW0 // 4, CPW1 // 4)
  qbase = jnp.where(cid == 0, sid * (CPW0 // 4),
                    NS * (CPW0 // 4) + sid * (CPW1 // 4))

  pltpu.sync_copy(iv_hbm.at[qbase], ib0)
  pltpu.async_copy(table_hbm.at[ib0.at[0]], rows_a, gs_a)

  def quad(q, ib, ib_nxt, is_nxt):
    not_last = q < nq - 1
    @pl.when(q > 0)
    def _():
      pltpu.make_async_copy(rows_b, psum_sp.at[ib.at[7]], ss_b).wait()
    @pl.when(not_last)
    def _():
      pltpu.async_copy(iv_hbm.at[qbase + q + 1], ib_nxt, is_nxt)
    pltpu.async_copy(table_hbm.at[ib.at[2]], rows_b, gs_b)
    pltpu.make_async_copy(table_hbm.at[ib.at[0]], rows_a, gs_a).wait()
    pltpu.async_copy(rows_a, psum_sp.at[ib.at[1]], ss_a, add=True)
    pltpu.make_async_copy(rows_a, psum_sp.at[ib.at[1]], ss_a).wait()
    pltpu.async_copy(table_hbm.at[ib.at[4]], rows_a, gs_a)
    pltpu.make_async_copy(table_hbm.at[ib.at[2]], rows_b, gs_b).wait()
    pltpu.async_copy(rows_b, psum_sp.at[ib.at[3]], ss_b, add=True)
    pltpu.make_async_copy(rows_b, psum_sp.at[ib.at[3]], ss_b).wait()
    pltpu.async_copy(table_hbm.at[ib.at[6]], rows_b, gs_b)
    pltpu.make_async_copy(table_hbm.at[ib.at[4]], rows_a, gs_a).wait()
    pltpu.async_copy(rows_a, psum_sp.at[ib.at[5]], ss_a, add=True)
    pltpu.make_async_copy(rows_a, psum_sp.at[ib.at[5]], ss_a).wait()
    @pl.when(not_last)
    def _():
      pltpu.make_async_copy(iv_hbm.at[qbase + q + 1], ib_nxt, is_nxt).wait()
      pltpu.async_copy(table_hbm.at[ib_nxt.at[0]], rows_a, gs_a)
    pltpu.make_async_copy(table_hbm.at[ib.at[6]], rows_b, gs_b).wait()
    pltpu.async_copy(rows_b, psum_sp.at[ib.at[7]], ss_b, add=True)

  def two_quads(h, carry):
    quad(2 * h, ib0, ib1, is1)
    @pl.when(2 * h + 1 < nq)
    def _():
      quad(2 * h + 1, ib1, ib0, is0)
    return carry
  lax.fori_loop(0, (nq + 1) // 2, two_quads, 0)

  last_ib = jnp.where(nq % 2 == 0, 0, 1)
  @pl.when(last_ib == 0)
  def _():
    pltpu.make_async_copy(rows_b, psum_sp.at[ib1.at[7]], ss_b).wait()
  @pl.when(last_ib == 1)
  def _():
    pltpu.make_async_copy(rows_b, psum_sp.at[ib0.at[7]], ss_b).wait()

  plsc.subcore_barrier()

  pltpu.sync_copy(psum_sp.at[pl.ds(base, RPT)],
                  out_sum.at[cid].at[pl.ds(base, RPT)])


_sc_segsum_call = pl.kernel(
    _sc_segsum_body, mesh=_MESH,
    out_type=jax.ShapeDtypeStruct((NC, N_PAD, D), jnp.float32),
    scratch_types=[
        pltpu.VMEM((8, C), jnp.int32),
        pltpu.VMEM((8, C), jnp.int32),
        pltpu.VMEM((C, D), jnp.float32),
        pltpu.VMEM((C, D), jnp.float32),
        pltpu.VMEM_SHARED((N_PAD, D), jnp.float32),
        pltpu.SemaphoreType.DMA,
        pltpu.SemaphoreType.DMA,
        pltpu.SemaphoreType.DMA,
        pltpu.SemaphoreType.DMA,
        pltpu.SemaphoreType.DMA,
        pltpu.SemaphoreType.DMA,
    ])


def _sc_count_body(dst_hbm, out_cnt, dst_v, ones_v, cnt_sp, sem):
  cid = lax.axis_index("c")
  sid = lax.axis_index("s")
  wid = cid * NS + sid
  base = sid * RPT

  _zero_fill(ones_v, 0.0)
  _zero_spmem_slice(ones_v, cnt_sp, base)
  _zero_fill(ones_v, 1.0)

  plsc.subcore_barrier()

  ebase = wid * (CPW * C)

  def step(j, carry):
    off = ebase + j * C
    pltpu.sync_copy(dst_hbm.at[pl.ds(off, C)], dst_v)
    pltpu.sync_copy(ones_v, cnt_sp.at[dst_v], add=True)
    return carry
  lax.fori_loop(0, CPW, step, 0)

  plsc.subcore_barrier()

  pltpu.sync_copy(cnt_sp.at[pl.ds(base, RPT)],
                  out_cnt.at[cid].at[pl.ds(base, RPT)])


_sc_count_call = pl.kernel(
    _sc_count_body, mesh=_MESH,
    out_type=jax.ShapeDtypeStruct((NC, N_PAD, D), jnp.float32),
    scratch_types=[
        pltpu.VMEM((C,), jnp.int32),
        pltpu.VMEM((C, D), jnp.float32),
        pltpu.VMEM_SHARED((N_PAD, D), jnp.float32),
        pltpu.SemaphoreType.DMA,
    ])

BR = 1000


def _tc_layer(p, cnt2, xin, w_rel, w_root, b, relu):
  def tc_body(p_ref, c_ref, x_ref, wr_ref, wt_ref, b_ref, o_ref):
    s = p_ref[0] + p_ref[1]
    cnt = c_ref[0, :, 0:1] + c_ref[1, :, 0:1]
    agg = s * (1.0 / jnp.maximum(cnt, 1.0))
    y = (jnp.dot(agg, wr_ref[...], preferred_element_type=jnp.float32)
         + jnp.dot(x_ref[...], wt_ref[...], preferred_element_type=jnp.float32)
         + b_ref[...])
    o_ref[...] = jnp.maximum(y, 0.0) if relu else y

  return pl.pallas_call(
      tc_body,
      grid=(N // BR,),
      in_specs=[
          pl.BlockSpec((NC, BR, D), lambda i: (0, i, 0)),
          pl.BlockSpec((NC, BR, D), lambda i: (0, i, 0)),
          pl.BlockSpec((BR, D), lambda i: (i, 0)),
          pl.BlockSpec((D, D), lambda i: (0, 0)),
          pl.BlockSpec((D, D), lambda i: (0, 0)),
          pl.BlockSpec((1, D), lambda i: (0, 0)),
      ],
      out_specs=pl.BlockSpec((BR, D), lambda i: (i, 0)),
      out_shape=jax.ShapeDtypeStruct((N, D), jnp.float32),
  )(p, cnt2, xin, w_rel, w_root, b)


@jax.jit
def kernel(x, edge_index, W1_rel, W1_root, b1, W2_rel, W2_root, b2):
  src = edge_index[0]
  dst = edge_index[1]
  pad = E_PAD - E
  src_p = jnp.concatenate([src, jnp.zeros((pad,), jnp.int32)])
  dst_p = jnp.concatenate([dst, jnp.full((pad,), N, jnp.int32)])

  s4 = src_p.reshape(-1, 4, C)
  d4 = dst_p.reshape(-1, 4, C)
  iv = jnp.stack([s4[:, 0], d4[:, 0], s4[:, 1], d4[:, 1],
                  s4[:, 2], d4[:, 2], s4[:, 3], d4[:, 3]], axis=1)

  cnt2 = _sc_count_call(dst_p)
  psum1 = _sc_segsum_call(x, iv)
  h = _tc_layer(psum1, cnt2, x, W1_rel, W1_root, b1.reshape(1, D), relu=True)
  psum2 = _sc_segsum_call(h, iv)
  out = _tc_layer(psum2, cnt2, h, W2_rel, W2_root, b2.reshape(1, D), relu=False)
  return out


# speedup vs baseline: 1.0330x; 1.0330x over previous
import functools

import jax
import jax.numpy as jnp
from jax import lax
from jax.experimental import pallas as pl
from jax.experimental.pallas import tpu as pltpu
from jax.experimental.pallas import tpu_sc as plsc

N = 10000
E = 320000
D = 128

NC = 2
NS = 16
NW = NC * NS
C = 128
CPW = 80
CPW0 = 104
CPW1 = 56
E_PAD = NW * CPW * C
N_PAD = 10112
RPT = N_PAD // NS

_MESH = plsc.VectorSubcoreMesh(core_axis_name="c", subcore_axis_name="s")


def _zero_fill(ref, val):
  rows, cols = ref.shape
  def body(i, carry):
    ref[i // (cols // 16), pl.ds((i % (cols // 16)) * 16, 16)] = (
        jnp.full((16,), val, jnp.float32))
    return carry
  lax.fori_loop(0, rows * (cols // 16), body, 0)


def _zero_spmem_slice(zsrc_v, sp, base):
  for k in range(RPT // C):
    pltpu.sync_copy(zsrc_v, sp.at[pl.ds(base + k * C, C)])
  rem = RPT % C
  if rem:
    pltpu.sync_copy(zsrc_v.at[pl.ds(0, rem)],
                    sp.at[pl.ds(base + (RPT // C) * C, rem)])


def _sc_segsum_body(table_hbm, iv_hbm, out_sum,
                    ib0, ib1, rows_a, rows_b,
                    psum_sp, gs_a, gs_b, ss_a, ss_b, is0, is1):
  cid = lax.axis_index("c")
  sid = lax.axis_index("s")
  base = sid * RPT

  _zero_fill(rows_a, 0.0)
  _zero_spmem_slice(rows_a, psum_sp, base)

  plsc.subcore_barrier()

  nq = jnp.where(cid == 0, CPW0 // 4, CPW1 // 4)
  qbase = jnp.where(cid == 0, sid * (CPW0 // 4),
                    NS * (CPW0 // 4) + sid * (CPW1 // 4))

  pltpu.sync_copy(iv_hbm.at[qbase], ib0)
  pltpu.async_copy(table_hbm.at[ib0.at[0]], rows_a, gs_a)

  def quad(q, ib, ib_nxt, is_nxt):
    not_last = q < nq - 1
    @pl.when(q > 0)
    def _():
      pltpu.make_async_copy(rows_b, psum_sp.at[ib.at[7]], ss_b).wait()
    @pl.when(not_last)
    def _():
      pltpu.async_copy(iv_hbm.at[qbase + q + 1], ib_nxt, is_nxt)
    pltpu.async_copy(table_hbm.at[ib.at[2]], rows_b, gs_b)
    pltpu.make_async_copy(table_hbm.at[ib.at[0]], rows_a, gs_a).wait()
    pltpu.async_copy(rows_a, psum_sp.at[ib.at[1]], ss_a, add=True)
    pltpu.make_async_copy(rows_a, psum_sp.at[ib.at[1]], ss_a).wait()
    pltpu.async_copy(table_hbm.at[ib.at[4]], rows_a, gs_a)
    pltpu.make_async_copy(table_hbm.at[ib.at[2]], rows_b, gs_b).wait()
    pltpu.async_copy(rows_b, psum_sp.at[ib.at[3]], ss_b, add=True)
    pltpu.make_async_copy(rows_b, psum_sp.at[ib.at[3]], ss_b).wait()
    pltpu.async_copy(table_hbm.at[ib.at[6]], rows_b, gs_b)
    pltpu.make_async_copy(table_hbm.at[ib.at[4]], rows_a, gs_a).wait()
    pltpu.async_copy(rows_a, psum_sp.at[ib.at[5]], ss_a, add=True)
    pltpu.make_async_copy(rows_a, psum_sp.at[ib.at[5]], ss_a).wait()
    @pl.when(not_last)
    def _():
      pltpu.make_async_copy(iv_hbm.at[qbase + q + 1], ib_nxt, is_nxt).wait()
      pltpu.async_copy(table_hbm.at[ib_nxt.at[0]], rows_a, gs_a)
    pltpu.make_async_copy(table_hbm.at[ib.at[6]], rows_b, gs_b).wait()
    pltpu.async_copy(rows_b, psum_sp.at[ib.at[7]], ss_b, add=True)

  def two_quads(h, carry):
    quad(2 * h, ib0, ib1, is1)
    @pl.when(2 * h + 1 < nq)
    def _():
      quad(2 * h + 1, ib1, ib0, is0)
    return carry
  lax.fori_loop(0, (nq + 1) // 2, two_quads, 0)

  last_ib = jnp.where(nq % 2 == 0, 0, 1)
  @pl.when(last_ib == 0)
  def _():
    pltpu.make_async_copy(rows_b, psum_sp.at[ib1.at[7]], ss_b).wait()
  @pl.when(last_ib == 1)
  def _():
    pltpu.make_async_copy(rows_b, psum_sp.at[ib0.at[7]], ss_b).wait()

  plsc.subcore_barrier()

  pltpu.sync_copy(psum_sp.at[pl.ds(base, RPT)],
                  out_sum.at[cid].at[pl.ds(base, RPT)])


_sc_segsum_call = pl.kernel(
    _sc_segsum_body, mesh=_MESH,
    out_type=jax.ShapeDtypeStruct((NC, N_PAD, D), jnp.float32),
    scratch_types=[
        pltpu.VMEM((8, C), jnp.int32),
        pltpu.VMEM((8, C), jnp.int32),
        pltpu.VMEM((C, D), jnp.float32),
        pltpu.VMEM((C, D), jnp.float32),
        pltpu.VMEM_SHARED((N_PAD, D), jnp.float32),
        pltpu.SemaphoreType.DMA,
        pltpu.SemaphoreType.DMA,
        pltpu.SemaphoreType.DMA,
        pltpu.SemaphoreType.DMA,
        pltpu.SemaphoreType.DMA,
        pltpu.SemaphoreType.DMA,
    ])


def _sc_count_body(dst_hbm, out_cnt, dst_a, dst_b, ones_v, cnt_sp,
                   ss_a, ss_b, is_a, is_b):
  cid = lax.axis_index("c")
  sid = lax.axis_index("s")
  wid = cid * NS + sid
  base = sid * RPT

  _zero_fill(ones_v, 0.0)
  _zero_spmem_slice(ones_v, cnt_sp, base)
  _zero_fill(ones_v, 1.0)

  plsc.subcore_barrier()

  ebase = wid * (CPW * C)
  npair = CPW // 2

  pltpu.async_copy(dst_hbm.at[pl.ds(ebase, C)], dst_a, is_a)

  def pair(g, carry):
    @pl.when(g > 0)
    def _():
      pltpu.make_async_copy(ones_v, cnt_sp.at[dst_b], ss_b).wait()
    pltpu.async_copy(dst_hbm.at[pl.ds(ebase + (2 * g + 1) * C, C)],
                     dst_b, is_b)
    pltpu.make_async_copy(dst_hbm.at[pl.ds(ebase, C)], dst_a, is_a).wait()
    pltpu.async_copy(ones_v, cnt_sp.at[dst_a], ss_a, add=True)
    pltpu.make_async_copy(dst_hbm.at[pl.ds(ebase, C)], dst_b, is_b).wait()
    pltpu.async_copy(ones_v, cnt_sp.at[dst_b], ss_b, add=True)
    pltpu.make_async_copy(ones_v, cnt_sp.at[dst_a], ss_a).wait()
    @pl.when(g < npair - 1)
    def _():
      pltpu.async_copy(dst_hbm.at[pl.ds(ebase + (2 * g + 2) * C, C)],
                       dst_a, is_a)
    return carry
  lax.fori_loop(0, npair, pair, 0)

  pltpu.make_async_copy(ones_v, cnt_sp.at[dst_b], ss_b).wait()

  plsc.subcore_barrier()

  pltpu.sync_copy(cnt_sp.at[pl.ds(base, RPT)],
                  out_cnt.at[cid].at[pl.ds(base, RPT)])


_sc_count_call = pl.kernel(
    _sc_count_body, mesh=_MESH,
    out_type=jax.ShapeDtypeStruct((NC, N_PAD, D), jnp.float32),
    scratch_types=[
        pltpu.VMEM((C,), jnp.int32),
        pltpu.VMEM((C,), jnp.int32),
        pltpu.VMEM((C, D), jnp.float32),
        pltpu.VMEM_SHARED((N_PAD, D), jnp.float32),
        pltpu.SemaphoreType.DMA,
        pltpu.SemaphoreType.DMA,
        pltpu.SemaphoreType.DMA,
        pltpu.SemaphoreType.DMA,
    ])

BR = 1000


def _tc_layer(p, cnt2, xin, w_rel, w_root, b, relu):
  def tc_body(p_ref, c_ref, x_ref, wr_ref, wt_ref, b_ref, o_ref):
    s = p_ref[0] + p_ref[1]
    cnt = c_ref[0, :, 0:1] + c_ref[1, :, 0:1]
    agg = s * (1.0 / jnp.maximum(cnt, 1.0))
    y = (jnp.dot(agg, wr_ref[...], preferred_element_type=jnp.float32)
         + jnp.dot(x_ref[...], wt_ref[...], preferred_element_type=jnp.float32)
         + b_ref[...])
    o_ref[...] = jnp.maximum(y, 0.0) if relu else y

  return pl.pallas_call(
      tc_body,
      grid=(N // BR,),
      in_specs=[
          pl.BlockSpec((NC, BR, D), lambda i: (0, i, 0)),
          pl.BlockSpec((NC, BR, D), lambda i: (0, i, 0)),
          pl.BlockSpec((BR, D), lambda i: (i, 0)),
          pl.BlockSpec((D, D), lambda i: (0, 0)),
          pl.BlockSpec((D, D), lambda i: (0, 0)),
          pl.BlockSpec((1, D), lambda i: (0, 0)),
      ],
      out_specs=pl.BlockSpec((BR, D), lambda i: (i, 0)),
      out_shape=jax.ShapeDtypeStruct((N, D), jnp.float32),
  )(p, cnt2, xin, w_rel, w_root, b)


@jax.jit
def kernel(x, edge_index, W1_rel, W1_root, b1, W2_rel, W2_root, b2):
  src = edge_index[0]
  dst = edge_index[1]
  pad = E_PAD - E
  src_p = jnp.concatenate([src, jnp.zeros((pad,), jnp.int32)])
  dst_p = jnp.concatenate([dst, jnp.full((pad,), N, jnp.int32)])

  s4 = src_p.reshape(-1, 4, C)
  d4 = dst_p.reshape(-1, 4, C)
  iv = jnp.stack([s4[:, 0], d4[:, 0], s4[:, 1], d4[:, 1],
                  s4[:, 2], d4[:, 2], s4[:, 3], d4[:, 3]], axis=1)

  cnt2 = _sc_count_call(dst_p)
  psum1 = _sc_segsum_call(x, iv)
  h = _tc_layer(psum1, cnt2, x, W1_rel, W1_root, b1.reshape(1, D), relu=True)
  psum2 = _sc_segsum_call(h, iv)
  out = _tc_layer(psum2, cnt2, h, W2_rel, W2_root, b2.reshape(1, D), relu=False)
  return out


# speedup vs baseline: 1.0334x; 1.0004x over previous
import jax
import jax.numpy as jnp
from jax import lax
from jax.experimental import pallas as pl
from jax.experimental.pallas import tpu as pltpu
from jax.experimental.pallas import tpu_sc as plsc

N = 10000
E = 320000
D = 128

NC = 2
NS = 16
NW = NC * NS
C = 128
CPW = 80
CPW0 = 104
CPW1 = 56
E_PAD = NW * CPW * C
N_PAD = 10112
RPT = N_PAD // NS

_MESH = plsc.VectorSubcoreMesh(core_axis_name="c", subcore_axis_name="s")


def _zero_fill(ref, val):
  rows, cols = ref.shape
  def body(i, carry):
    ref[i // (cols // 16), pl.ds((i % (cols // 16)) * 16, 16)] = (
        jnp.full((16,), val, jnp.float32))
    return carry
  lax.fori_loop(0, rows * (cols // 16), body, 0)


def _zero_spmem_slice(zsrc_v, sp, base):
  for k in range(RPT // C):
    pltpu.sync_copy(zsrc_v, sp.at[pl.ds(base + k * C, C)])
  rem = RPT % C
  if rem:
    pltpu.sync_copy(zsrc_v.at[pl.ds(0, rem)],
                    sp.at[pl.ds(base + (RPT // C) * C, rem)])


def _sc_segsum_body(table_hbm, iv_hbm, out_sum,
                    ib0, ib1, rows_a, rows_b,
                    psum_sp, gs_a, gs_b, ss_a, ss_b, is0, is1):
  cid = lax.axis_index("c")
  sid = lax.axis_index("s")
  base = sid * RPT

  _zero_fill(rows_a, 0.0)
  _zero_spmem_slice(rows_a, psum_sp, base)

  plsc.subcore_barrier()

  nq = jnp.where(cid == 0, CPW0 // 4, CPW1 // 4)
  qbase = jnp.where(cid == 0, sid * (CPW0 // 4),
                    NS * (CPW0 // 4) + sid * (CPW1 // 4))

  pltpu.sync_copy(iv_hbm.at[qbase], ib0)
  pltpu.async_copy(table_hbm.at[ib0.at[0]], rows_a, gs_a)

  def quad(q, ib, ib_nxt, is_nxt):
    not_last = q < nq - 1
    @pl.when(q > 0)
    def _():
      pltpu.make_async_copy(rows_b, psum_sp.at[ib.at[7]], ss_b).wait()
    @pl.when(not_last)
    def _():
      pltpu.async_copy(iv_hbm.at[qbase + q + 1], ib_nxt, is_nxt)
    pltpu.async_copy(table_hbm.at[ib.at[2]], rows_b, gs_b)
    pltpu.make_async_copy(table_hbm.at[ib.at[0]], rows_a, gs_a).wait()
    pltpu.async_copy(rows_a, psum_sp.at[ib.at[1]], ss_a, add=True)
    pltpu.make_async_copy(rows_a, psum_sp.at[ib.at[1]], ss_a).wait()
    pltpu.async_copy(table_hbm.at[ib.at[4]], rows_a, gs_a)
    pltpu.make_async_copy(table_hbm.at[ib.at[2]], rows_b, gs_b).wait()
    pltpu.async_copy(rows_b, psum_sp.at[ib.at[3]], ss_b, add=True)
    pltpu.make_async_copy(rows_b, psum_sp.at[ib.at[3]], ss_b).wait()
    pltpu.async_copy(table_hbm.at[ib.at[6]], rows_b, gs_b)
    pltpu.make_async_copy(table_hbm.at[ib.at[4]], rows_a, gs_a).wait()
    pltpu.async_copy(rows_a, psum_sp.at[ib.at[5]], ss_a, add=True)
    pltpu.make_async_copy(rows_a, psum_sp.at[ib.at[5]], ss_a).wait()
    @pl.when(not_last)
    def _():
      pltpu.make_async_copy(iv_hbm.at[qbase + q + 1], ib_nxt, is_nxt).wait()
      pltpu.async_copy(table_hbm.at[ib_nxt.at[0]], rows_a, gs_a)
    pltpu.make_async_copy(table_hbm.at[ib.at[6]], rows_b, gs_b).wait()
    pltpu.async_copy(rows_b, psum_sp.at[ib.at[7]], ss_b, add=True)

  def two_quads(h, carry):
    quad(2 * h, ib0, ib1, is1)
    @pl.when(2 * h + 1 < nq)
    def _():
      quad(2 * h + 1, ib1, ib0, is0)
    return carry
  lax.fori_loop(0, (nq + 1) // 2, two_quads, 0)

  last_ib = jnp.where(nq % 2 == 0, 0, 1)
  @pl.when(last_ib == 0)
  def _():
    pltpu.make_async_copy(rows_b, psum_sp.at[ib1.at[7]], ss_b).wait()
  @pl.when(last_ib == 1)
  def _():
    pltpu.make_async_copy(rows_b, psum_sp.at[ib0.at[7]], ss_b).wait()

  plsc.subcore_barrier()

  pltpu.sync_copy(psum_sp.at[pl.ds(base, RPT)],
                  out_sum.at[cid].at[pl.ds(base, RPT)])


_sc_segsum_call = pl.kernel(
    _sc_segsum_body, mesh=_MESH,
    out_type=jax.ShapeDtypeStruct((NC, N_PAD, D), jnp.float32),
    scratch_types=[
        pltpu.VMEM((8, C), jnp.int32),
        pltpu.VMEM((8, C), jnp.int32),
        pltpu.VMEM((C, D), jnp.float32),
        pltpu.VMEM((C, D), jnp.float32),
        pltpu.VMEM_SHARED((N_PAD, D), jnp.float32),
        pltpu.SemaphoreType.DMA,
        pltpu.SemaphoreType.DMA,
        pltpu.SemaphoreType.DMA,
        pltpu.SemaphoreType.DMA,
        pltpu.SemaphoreType.DMA,
        pltpu.SemaphoreType.DMA,
    ])


def _sc_count_body(dst_hbm, out_cnt, dst_a, dst_b, ones_v, cnt_sp,
                   ss_a, ss_b, is_a, is_b):
  cid = lax.axis_index("c")
  sid = lax.axis_index("s")
  wid = cid * NS + sid
  base = sid * RPT

  _zero_fill(ones_v, 0.0)
  _zero_spmem_slice(ones_v, cnt_sp, base)
  _zero_fill(ones_v, 1.0)

  plsc.subcore_barrier()

  ebase = wid * (CPW * C)
  npair = CPW // 2

  pltpu.async_copy(dst_hbm.at[pl.ds(ebase, C)], dst_a, is_a)

  def pair(g, carry):
    @pl.when(g > 0)
    def _():
      pltpu.make_async_copy(ones_v, cnt_sp.at[dst_b], ss_b).wait()
    pltpu.async_copy(dst_hbm.at[pl.ds(ebase + (2 * g + 1) * C, C)],
                     dst_b, is_b)
    pltpu.make_async_copy(dst_hbm.at[pl.ds(ebase, C)], dst_a, is_a).wait()
    pltpu.async_copy(ones_v, cnt_sp.at[dst_a], ss_a, add=True)
    pltpu.make_async_copy(dst_hbm.at[pl.ds(ebase, C)], dst_b, is_b).wait()
    pltpu.async_copy(ones_v, cnt_sp.at[dst_b], ss_b, add=True)
    pltpu.make_async_copy(ones_v, cnt_sp.at[dst_a], ss_a).wait()
    @pl.when(g < npair - 1)
    def _():
      pltpu.async_copy(dst_hbm.at[pl.ds(ebase + (2 * g + 2) * C, C)],
                       dst_a, is_a)
    return carry
  lax.fori_loop(0, npair, pair, 0)

  pltpu.make_async_copy(ones_v, cnt_sp.at[dst_b], ss_b).wait()

  plsc.subcore_barrier()

  pltpu.sync_copy(cnt_sp.at[pl.ds(base, RPT)],
                  out_cnt.at[cid].at[pl.ds(base, RPT)])


_sc_count_call = pl.kernel(
    _sc_count_body, mesh=_MESH,
    out_type=jax.ShapeDtypeStruct((NC, N_PAD, D), jnp.float32),
    scratch_types=[
        pltpu.VMEM((C,), jnp.int32),
        pltpu.VMEM((C,), jnp.int32),
        pltpu.VMEM((C, D), jnp.float32),
        pltpu.VMEM_SHARED((N_PAD, D), jnp.float32),
        pltpu.SemaphoreType.DMA,
        pltpu.SemaphoreType.DMA,
        pltpu.SemaphoreType.DMA,
        pltpu.SemaphoreType.DMA,
    ])

BR = 1000


def _tc_layer(p, cnt2, xin, w_rel, w_root, b, relu):
  def tc_body(p_ref, c_ref, x_ref, wr_ref, wt_ref, b_ref, o_ref):
    s = p_ref[0] + p_ref[1]
    cnt = c_ref[0, :, 0:1] + c_ref[1, :, 0:1]
    agg = s * (1.0 / jnp.maximum(cnt, 1.0))
    y = (jnp.dot(agg, wr_ref[...], preferred_element_type=jnp.float32)
         + jnp.dot(x_ref[...], wt_ref[...], preferred_element_type=jnp.float32)
         + b_ref[...])
    o_ref[...] = jnp.maximum(y, 0.0) if relu else y

  return pl.pallas_call(
      tc_body,
      grid=(N // BR,),
      in_specs=[
          pl.BlockSpec((NC, BR, D), lambda i: (0, i, 0)),
          pl.BlockSpec((NC, BR, D), lambda i: (0, i, 0)),
          pl.BlockSpec((BR, D), lambda i: (i, 0)),
          pl.BlockSpec((D, D), lambda i: (0, 0)),
          pl.BlockSpec((D, D), lambda i: (0, 0)),
          pl.BlockSpec((1, D), lambda i: (0, 0)),
      ],
      out_specs=pl.BlockSpec((BR, D), lambda i: (i, 0)),
      out_shape=jax.ShapeDtypeStruct((N, D), jnp.float32),
  )(p, cnt2, xin, w_rel, w_root, b)


@jax.jit
def kernel(x, edge_index, W1_rel, W1_root, b1, W2_rel, W2_root, b2):
  src = edge_index[0]
  dst = edge_index[1]
  pad = E_PAD - E
  src_p = jnp.concatenate([src, jnp.zeros((pad,), jnp.int32)])
  dst_p = jnp.concatenate([dst, jnp.full((pad,), N, jnp.int32)])

  s4 = src_p.reshape(-1, 4, C)
  d4 = dst_p.reshape(-1, 4, C)
  iv = jnp.stack([s4[:, 0], d4[:, 0], s4[:, 1], d4[:, 1],
                  s4[:, 2], d4[:, 2], s4[:, 3], d4[:, 3]], axis=1)

  cnt2 = _sc_count_call(dst_p)
  psum1 = _sc_segsum_call(x, iv)
  h = _tc_layer(psum1, cnt2, x, W1_rel, W1_root, b1.reshape(1, D), relu=True)
  psum2 = _sc_segsum_call(h, iv)
  out = _tc_layer(psum2, cnt2, h, W2_rel, W2_root, b2.reshape(1, D), relu=False)
  return out
